# Initial kernel scaffold; baseline (speedup 1.0000x reference)
import jax
import jax.numpy as jnp
from jax.experimental import pallas as pl


def kernel(unique_params, index_map):
    raise NotImplementedError("write your pallas kernel here")


# speedup vs baseline: 1.1884x; 1.1884x over previous
import functools

import jax
import jax.numpy as jnp
from jax import lax
from jax.experimental import pallas as pl
from jax.experimental.pallas import tpu as pltpu
from jax.experimental.pallas import tpu_sc as plsc

H = W = 14
HW = H * W
B = HW * HW
D = 32 * 32
V = (2 * H - 1) * (2 * W - 1)

T = 56
NT = B // T
NW = 32
NITER = (NT + NW - 1) // NW


def _make_gather():
    mesh = plsc.VectorSubcoreMesh(core_axis_name="c", subcore_axis_name="s")

    @functools.partial(
        pl.kernel,
        mesh=mesh,
        out_type=jax.ShapeDtypeStruct((B, D), jnp.float32),
        scratch_types=[
            pltpu.VMEM((T,), jnp.int32),
            pltpu.VMEM((T, D), jnp.float32),
            pltpu.SemaphoreType.DMA,
        ],
    )
    def gather_kernel(table_hbm, idx_hbm, out_hbm, idx_v, rows_v, sem):
        wid = lax.axis_index("s") * 2 + lax.axis_index("c")

        def body(i, _):
            t = wid + i * NW

            @pl.when(t < NT)
            def _():
                base = t * T
                pltpu.sync_copy(idx_hbm.at[pl.ds(base, T)], idx_v)
                pltpu.async_copy(table_hbm.at[idx_v], rows_v, sem).wait()
                pltpu.sync_copy(rows_v, out_hbm.at[pl.ds(base, T)])

            return _

        lax.fori_loop(0, NITER, body, None)

    return gather_kernel


_gather = _make_gather()


def kernel(unique_params, index_map):
    table = unique_params.reshape(V, D)
    idx = index_map.reshape(B).astype(jnp.int32)
    out = _gather(table, idx)
    return out.reshape(HW, HW, 32, 32)


# speedup vs baseline: 1.2130x; 1.0207x over previous
import functools

import jax
import jax.numpy as jnp
from jax import lax
from jax.experimental import pallas as pl
from jax.experimental.pallas import tpu as pltpu
from jax.experimental.pallas import tpu_sc as plsc

H = W = 14
HW = H * W
B = HW * HW
D = 32 * 32
V = (2 * H - 1) * (2 * W - 1)

NW = 32
C = 1216
STRIDE = 1200
T = 32
NTW = C // T


def _make_gather():
    mesh = plsc.VectorSubcoreMesh(core_axis_name="c", subcore_axis_name="s")

    @functools.partial(
        pl.kernel,
        mesh=mesh,
        out_type=jax.ShapeDtypeStruct((B, D), jnp.float32),
        scratch_types=[
            pltpu.VMEM((C,), jnp.int32),
            pltpu.VMEM((2, T, D), jnp.float32),
            pltpu.SemaphoreType.DMA,
            pltpu.SemaphoreType.DMA,
        ],
    )
    def gather_kernel(table_hbm, idx_hbm, out_hbm, idx_v, rows_v, gsem, ssem):
        wid = lax.axis_index("s") * 2 + lax.axis_index("c")
        base = wid * STRIDE

        pltpu.sync_copy(idx_hbm.at[pl.ds(base, C)], idx_v)

        def gather_start(i, phase):
            return pltpu.async_copy(
                table_hbm.at[idx_v.at[pl.ds(i * T, T)]], rows_v.at[phase], gsem)

        def scatter_start(i, phase):
            return pltpu.async_copy(
                rows_v.at[phase], out_hbm.at[pl.ds(base + i * T, T)], ssem)

        def gather_drain(phase):
            pltpu.make_async_copy(
                table_hbm.at[pl.ds(0, T)], rows_v.at[phase], gsem).wait()

        def scatter_drain(phase):
            pltpu.make_async_copy(
                rows_v.at[phase], out_hbm.at[pl.ds(base, T)], ssem).wait()

        gather_start(0, 0)
        gather_drain(0)
        scatter_start(0, 0)
        gather_start(1, 1)

        def body(i, _):
            phase = lax.rem(i, 2)
            gather_drain(phase)
            scatter_start(i, phase)
            scatter_drain(1 - phase)

            @pl.when(i + 1 < NTW)
            def _():
                gather_start(i + 1, 1 - phase)

            return _

        lax.fori_loop(1, NTW, body, None)
        scatter_drain(1)

    return gather_kernel


_gather = _make_gather()


def kernel(unique_params, index_map):
    table = unique_params.reshape(V, D)
    idx = index_map.reshape(B).astype(jnp.int32)
    out = _gather(table, idx)
    return out.reshape(HW, HW, 32, 32)


# speedup vs baseline: 2.1838x; 1.8004x over previous
import functools

import jax
import jax.numpy as jnp
from jax import lax
from jax.experimental import pallas as pl
from jax.experimental.pallas import tpu as pltpu
from jax.experimental.pallas import tpu_sc as plsc

H = W = 14
HW = H * W
V = (2 * H - 1) * (2 * W - 1)
IO = 32
JP = 208
NJV = JP // 16


def _make_gather():
    mesh = plsc.VectorSubcoreMesh(core_axis_name="c", subcore_axis_name="s")

    @functools.partial(
        pl.kernel,
        mesh=mesh,
        out_type=jax.ShapeDtypeStruct((HW, IO, IO, HW), jnp.float32),
        compiler_params=pltpu.CompilerParams(needs_layout_passes=False),
        scratch_types=[
            pltpu.VMEM((HW * JP,), jnp.int32),
            pltpu.VMEM((IO * V,), jnp.float32),
            pltpu.VMEM((2, IO, HW), jnp.float32),
            pltpu.SemaphoreType.DMA,
        ],
    )
    def gather_kernel(tabt_hbm, idxp_hbm, out_hbm, idx_v, tab_v, buf_v, sem):
        w = lax.axis_index("s") * 2 + lax.axis_index("c")

        pltpu.sync_copy(idxp_hbm, idx_v)
        pltpu.sync_copy(tabt_hbm.at[w], tab_v)

        def drain_one():
            pltpu.make_async_copy(
                buf_v.at[0], out_hbm.at[0, w], sem).wait()

        def body(i, _):
            ph = lax.rem(i, 2)

            @pl.when(i >= 2)
            def _():
                drain_one()

            offs = [jv * 16 for jv in range(NJV - 1)] + [HW - 16]
            ivs = [idx_v[pl.ds(i * JP + o, 16)] for o in offs]

            def out_body(o, carry):
                ov = jnp.full((16,), o * V, jnp.int32)
                for jv, joff in enumerate(offs):
                    vals = plsc.load_gather(tab_v, [ov + ivs[jv]])
                    buf_v[ph, o, pl.ds(joff, 16)] = vals
                return carry

            lax.fori_loop(0, IO, out_body, 0)
            pltpu.async_copy(buf_v.at[ph], out_hbm.at[i, w], sem)
            return _

        lax.fori_loop(0, HW, body, None)
        drain_one()
        drain_one()

    return gather_kernel


_gather = _make_gather()


def kernel(unique_params, index_map):
    tabt = unique_params.transpose(1, 2, 0).reshape(IO, IO * V)
    idxp = jnp.pad(index_map, ((0, 0), (0, JP - HW))).reshape(-1)
    out = _gather(tabt, idxp.astype(jnp.int32))
    return out.transpose(0, 3, 1, 2)


# speedup vs baseline: 8.3573x; 3.8270x over previous
import functools

import jax
import jax.numpy as jnp
from jax import lax
from jax.experimental import pallas as pl
from jax.experimental.pallas import tpu as pltpu
from jax.experimental.pallas import tpu_sc as plsc

H = W = 14
HW = H * W
V = (2 * H - 1) * (2 * W - 1)
IO = 32
JP = 208
NJV = JP // 16


def _make_gather():
    mesh = plsc.VectorSubcoreMesh(core_axis_name="c", subcore_axis_name="s")

    @functools.partial(
        pl.kernel,
        mesh=mesh,
        out_type=jax.ShapeDtypeStruct((HW, IO, IO, HW), jnp.float32),
        compiler_params=pltpu.CompilerParams(needs_layout_passes=False),
        scratch_types=[
            pltpu.VMEM((HW * JP,), jnp.int32),
            pltpu.VMEM((IO * V,), jnp.float32),
            pltpu.VMEM((2, IO, HW), jnp.float32),
            pltpu.SemaphoreType.DMA,
        ],
    )
    def gather_kernel(tabt_hbm, idxp_hbm, out_hbm, idx_v, tab_v, buf_v, sem):
        w = lax.axis_index("s") * 2 + lax.axis_index("c")

        pltpu.sync_copy(idxp_hbm, idx_v)
        pltpu.sync_copy(tabt_hbm.at[w], tab_v)

        def drain_one():
            pltpu.make_async_copy(
                buf_v.at[0], out_hbm.at[0, w], sem).wait()

        def body(i, _):
            ph = lax.rem(i, 2)

            @pl.when(i >= 2)
            def _():
                drain_one()

            offs = [jv * 16 for jv in range(NJV - 1)] + [HW - 16]
            ivs = [idx_v[pl.ds(i * JP + o, 16)] for o in offs]

            @plsc.parallel_loop(0, IO, unroll=2)
            def out_body(o):
                ov = jnp.full((16,), o * V, jnp.int32)
                for jv, joff in enumerate(offs):
                    vals = plsc.load_gather(tab_v, [ov + ivs[jv]])
                    buf_v[ph, o, pl.ds(joff, 16)] = vals
            pltpu.async_copy(buf_v.at[ph], out_hbm.at[i, w], sem)
            return _

        lax.fori_loop(0, HW, body, None)
        drain_one()
        drain_one()

    return gather_kernel


_gather = _make_gather()


def kernel(unique_params, index_map):
    tabt = unique_params.transpose(1, 2, 0).reshape(IO, IO * V)
    idxp = jnp.pad(index_map, ((0, 0), (0, JP - HW))).reshape(-1)
    out = _gather(tabt, idxp.astype(jnp.int32))
    return out.transpose(0, 3, 1, 2)
